# baseline (device time: 65380 ns/iter reference)
import jax
import jax.numpy as jnp
from jax import lax
from jax.experimental import pallas as pl
from jax.experimental.pallas import tpu as pltpu

N_DEV = 8
PART_COLS = (768, 640, 640)
PART_OFF = (0, 768, 1408)
DIMS = ((0, 1, 2), (1, 2, 0), (2, 0, 1))


def kernel(x, w_mat):
    m_global, k_shard = x.shape
    _, n = w_mat.shape
    m_per = m_global // N_DEV

    def body(x_ref, w_ref, out_ref,
             acc0, acc1, acc2, rcv0, rcv1, rcv2,
             send_sems, recv_sems):
        accs = (acc0, acc1, acc2)
        rcvs = (rcv0, rcv1, rcv2)

        i = lax.axis_index("i")
        mz = i // 4
        j = lax.rem(i, 4)
        my_ = j // 2
        mx = my_ ^ lax.rem(j, 2)
        me = (mx, my_, mz)

        def ring(cx, cy, cz):
            return 4 * cz + 2 * cy + (cx ^ cy)

        def flipped(d):
            c = list(me)
            c[d] = 1 - c[d]
            return ring(*c)

        partner = [[flipped(DIMS[p][r]) for r in range(3)] for p in range(3)]

        barrier = pltpu.get_barrier_semaphore()
        for d in range(3):
            pl.semaphore_signal(barrier, inc=1, device_id=(flipped(d),),
                                device_id_type=pl.DeviceIdType.MESH)
        pl.semaphore_wait(barrier, 3)

        def chunk_of(p, t):
            bits = [(t >> 2) & 1, (t >> 1) & 1, t & 1]
            c = list(me)
            for k in range(3):
                if bits[k]:
                    c[DIMS[p][k]] = 1 - c[DIMS[p][k]]
            return ring(*c)

        def gemm(p, t):
            c = chunk_of(p, t)
            xc = x_ref[pl.ds(c * m_per, m_per), :]
            wc = w_ref[:, PART_OFF[p]:PART_OFF[p] + PART_COLS[p]]
            return lax.dot_general(
                xc, wc,
                dimension_numbers=(((1,), (0,)), ((), ())),
                preferred_element_type=jnp.float32,
            )

        def sem_idx(p, r, b):
            return p * 7 + (0, 4, 6)[r] + b

        def make(p, r, b):
            if r == 0:
                src = slice(7 - b, 8 - b)
                dst = slice(3 - b, 4 - b)
            elif r == 1:
                src = slice(3 - b, 4 - b)
                dst = slice(5 - b, 6 - b)
            else:
                src = slice(1, 2)
                dst = slice(6, 7)
            k = sem_idx(p, r, b)
            return pltpu.make_async_remote_copy(
                src_ref=accs[p].at[src],
                dst_ref=rcvs[p].at[dst],
                send_sem=send_sems.at[k],
                recv_sem=recv_sems.at[k],
                device_id=(partner[p][r],),
                device_id_type=pl.DeviceIdType.MESH,
            )

        rdmas = {}

        def launch(p, r, b):
            rd = make(p, r, b)
            rd.start()
            rdmas[(p, r, b)] = rd

        for b in range(4):
            for p in range(3):
                accs[p][7 - b, :, :] = gemm(p, 7 - b)
                launch(p, 0, b)
        for t in (3, 2, 1, 0):
            for p in range(3):
                accs[p][t, :, :] = gemm(p, t)

        for p in range(3):
            rdmas[(p, 0, 0)].wait_recv()
            accs[p][3, :, :] = accs[p][3, :, :] + rcvs[p][3, :, :]
            launch(p, 1, 0)
        for p in range(3):
            rdmas[(p, 0, 1)].wait_recv()
            accs[p][2, :, :] = accs[p][2, :, :] + rcvs[p][2, :, :]
            launch(p, 1, 1)
        for p in range(3):
            rdmas[(p, 0, 2)].wait_recv()
            accs[p][1, :, :] = accs[p][1, :, :] + rcvs[p][1, :, :]
        for p in range(3):
            rdmas[(p, 1, 0)].wait_recv()
            accs[p][1, :, :] = accs[p][1, :, :] + rcvs[p][5, :, :]
            launch(p, 2, 0)
        for p in range(3):
            rdmas[(p, 0, 3)].wait_recv()
            accs[p][0, :, :] = accs[p][0, :, :] + rcvs[p][0, :, :]
        for p in range(3):
            rdmas[(p, 1, 1)].wait_recv()
            accs[p][0, :, :] = accs[p][0, :, :] + rcvs[p][4, :, :]
        for p in range(3):
            rdmas[(p, 2, 0)].wait_recv()
            out_ref[:, PART_OFF[p]:PART_OFF[p] + PART_COLS[p]] = jnp.maximum(
                accs[p][0, :, :] + rcvs[p][6, :, :], 0.0)

        for key in rdmas:
            rdmas[key].wait_send()

    return pl.pallas_call(
        body,
        out_shape=jax.ShapeDtypeStruct((m_per, n), jnp.float32),
        in_specs=[
            pl.BlockSpec(memory_space=pltpu.VMEM),
            pl.BlockSpec(memory_space=pltpu.VMEM),
        ],
        out_specs=pl.BlockSpec(memory_space=pltpu.VMEM),
        scratch_shapes=[
            pltpu.VMEM((N_DEV, m_per, PART_COLS[0]), jnp.float32),
            pltpu.VMEM((N_DEV, m_per, PART_COLS[1]), jnp.float32),
            pltpu.VMEM((N_DEV, m_per, PART_COLS[2]), jnp.float32),
            pltpu.VMEM((7, m_per, PART_COLS[0]), jnp.float32),
            pltpu.VMEM((7, m_per, PART_COLS[1]), jnp.float32),
            pltpu.VMEM((7, m_per, PART_COLS[2]), jnp.float32),
            pltpu.SemaphoreType.DMA((21,)),
            pltpu.SemaphoreType.DMA((21,)),
        ],
        compiler_params=pltpu.CompilerParams(collective_id=0),
    )(x, w_mat)
